# Initial kernel scaffold; baseline (speedup 1.0000x reference)
import jax
import jax.numpy as jnp
from jax.experimental import pallas as pl


def kernel(x, W_l0, b_l0, W_r0, W_l1, b_l1, W_r1, W_l2, b_l2, W_r2, Wp0, bp0, Wp2, bp2, g0, be0, g1, be1, g2, be2, edge_index, batch):
    raise NotImplementedError("write your pallas kernel here")


# speedup vs baseline: 1.8286x; 1.8286x over previous
import functools

import jax
import jax.numpy as jnp
from jax import lax
from jax.experimental import pallas as pl
from jax.experimental.pallas import tpu as pltpu
from jax.experimental.pallas import tpu_sc as plsc

N = 10000
E = 160000
NG = 64
NTILES = 32
EDGES_PER_TILE = 5120
NBATCH = 40
K = 128
NROWS = 10240
ROWS_PER_SUB = NROWS // 16
DUMP_ROW = N
BLK = 1000
GRID = N // BLK

_f32 = jnp.float32


def _dotT(a, w):
    return lax.dot_general(a, w, (((1,), (1,)), ((), ())),
                           precision=lax.Precision.HIGHEST,
                           preferred_element_type=_f32)


def _make_segsum(nch):
    C = 128
    mesh = plsc.VectorSubcoreMesh(core_axis_name="c", subcore_axis_name="s")
    scratch = [
        pltpu.VMEM((NBATCH, K), jnp.int32),
        pltpu.VMEM((NBATCH, K), jnp.int32),
        pltpu.VMEM((K, C), _f32),
        pltpu.VMEM((16, C), _f32),
        pltpu.VMEM_SHARED((NROWS, C), _f32),
        pltpu.SemaphoreType.DMA,
    ]

    def body(*refs):
        xs = refs[:nch]
        (src_hbm, dst_hbm, out_hbm,
         src_v, dst_v, rows_v, zbuf, acc, sem) = refs[nch:]
        cid = lax.axis_index("c")
        sid = lax.axis_index("s")
        wid = cid * 16 + sid
        pltpu.sync_copy(src_hbm.at[wid], src_v)
        pltpu.sync_copy(dst_hbm.at[wid], dst_v)
        for j in range(16):
            for k2 in range(C // 16):
                zbuf[j, pl.ds(k2 * 16, 16)] = jnp.zeros((16,), _f32)
        base = sid * ROWS_PER_SUB
        for c in range(nch):
            def zrow(r, carry):
                pltpu.sync_copy(zbuf, acc.at[pl.ds(base + r * 16, 16)])
                return carry
            lax.fori_loop(0, ROWS_PER_SUB // 16, zrow, 0)
            plsc.subcore_barrier()

            def step(b, carry):
                pltpu.async_copy(xs[c].at[src_v.at[b]], rows_v, sem).wait()
                pltpu.sync_copy(rows_v, acc.at[dst_v.at[b]], add=True)
                return carry
            lax.fori_loop(0, NBATCH, step, 0)
            plsc.subcore_barrier()
            pltpu.sync_copy(acc.at[pl.ds(base, ROWS_PER_SUB)],
                            out_hbm.at[c, cid, pl.ds(base, ROWS_PER_SUB)])
            plsc.subcore_barrier()

    return pl.kernel(
        body,
        out_type=jax.ShapeDtypeStruct((nch, 2, NROWS, 128), _f32),
        mesh=mesh,
        scratch_types=scratch,
    )


def _stats_update(st_ref, y, i):
    srow = jnp.sum(y, axis=0, keepdims=True)
    sq = jnp.sum(y * y, axis=0, keepdims=True)
    upd = jnp.concatenate(
        [srow, sq, jnp.zeros((6, y.shape[1]), _f32)], axis=0)

    @pl.when(i == 0)
    def _():
        st_ref[...] = upd

    @pl.when(i > 0)
    def _():
        st_ref[...] = st_ref[...] + upd


def _dinv_col(d_ref):
    dcol = (d_ref[0] + d_ref[1])[:, :1]
    return 1.0 / jnp.maximum(dcol, 1.0)


def _sage_a_body(nch, with_skip, a_ref, d_ref, x_ref, wl_ref, wr_ref, *rest):
    if with_skip:
        wp_ref, y_ref, s_ref, st_ref = rest
    else:
        y_ref, st_ref = rest
    i = pl.program_id(0)
    parts = [a_ref[c, 0] + a_ref[c, 1] for c in range(nch)]
    p = parts[0] if nch == 1 else jnp.concatenate(parts, axis=-1)
    mean = p * _dinv_col(d_ref)
    xb = x_ref[...]
    y = _dotT(mean, wl_ref[...]) + _dotT(xb, wr_ref[...])
    y_ref[...] = y
    if with_skip:
        s_ref[...] = _dotT(xb, wp_ref[...])
    _stats_update(st_ref, y, i)


def _sage_a(nch, Din, Dout, with_skip):
    a_spec = pl.BlockSpec((nch, 2, BLK, 128), lambda i: (0, 0, i, 0))
    d_spec = pl.BlockSpec((2, BLK, 128), lambda i: (0, i, 0))
    x_spec = pl.BlockSpec((BLK, Din), lambda i: (i, 0))
    w_spec = pl.BlockSpec((Dout, Din), lambda i: (0, 0))
    y_spec = pl.BlockSpec((BLK, Dout), lambda i: (i, 0))
    st_spec = pl.BlockSpec((8, Dout), lambda i: (0, 0))
    in_specs = [a_spec, d_spec, x_spec, w_spec, w_spec]
    out_shapes = [jax.ShapeDtypeStruct((N, Dout), _f32),
                  jax.ShapeDtypeStruct((8, Dout), _f32)]
    out_specs = [y_spec, st_spec]
    if with_skip:
        in_specs.append(w_spec)
        out_shapes.insert(1, jax.ShapeDtypeStruct((N, Dout), _f32))
        out_specs.insert(1, y_spec)
    return pl.pallas_call(
        functools.partial(_sage_a_body, nch, with_skip),
        grid=(GRID,),
        in_specs=in_specs,
        out_specs=out_specs,
        out_shape=out_shapes,
    )


def _bn_b_body(relu, y_ref, s_ref, st_ref, p_ref, o_ref):
    st = st_ref[...]
    m = st[0:1] / float(N)
    var = st[1:2] / float(N) - m * m
    scale = p_ref[0:1] * lax.rsqrt(var + 1e-5)
    h = (y_ref[...] - m) * scale + p_ref[1:2] + s_ref[...] + p_ref[2:3]
    o_ref[...] = jnp.maximum(h, 0.0) if relu else h


def _bn_b(Dout, relu=True):
    y_spec = pl.BlockSpec((BLK, Dout), lambda i: (i, 0))
    st_spec = pl.BlockSpec((8, Dout), lambda i: (0, 0))
    return pl.pallas_call(
        functools.partial(_bn_b_body, relu),
        grid=(GRID,),
        in_specs=[y_spec, y_spec, st_spec, st_spec],
        out_specs=y_spec,
        out_shape=jax.ShapeDtypeStruct((N, Dout), _f32),
    )


def _l2pre_body(h_ref, wl_ref, wr_ref, wp_ref, yl_ref, yr_ref, s_ref):
    hb = h_ref[...]
    yl_ref[...] = _dotT(hb, wl_ref[...])
    yr_ref[...] = _dotT(hb, wr_ref[...])
    s_ref[...] = _dotT(hb, wp_ref[...])


def _l2pre():
    h_spec = pl.BlockSpec((BLK, 512), lambda i: (i, 0))
    w_spec = pl.BlockSpec((128, 512), lambda i: (0, 0))
    o_spec = pl.BlockSpec((BLK, 128), lambda i: (i, 0))
    return pl.pallas_call(
        _l2pre_body,
        grid=(GRID,),
        in_specs=[h_spec, w_spec, w_spec, w_spec],
        out_specs=[o_spec, o_spec, o_spec],
        out_shape=[jax.ShapeDtypeStruct((N, 128), _f32)] * 3,
    )


def _l2fin_body(a_ref, d_ref, yr_ref, y_ref, st_ref):
    i = pl.program_id(0)
    y = (a_ref[0, 0] + a_ref[0, 1]) * _dinv_col(d_ref) + yr_ref[...]
    y_ref[...] = y
    _stats_update(st_ref, y, i)


def _l2fin():
    a_spec = pl.BlockSpec((1, 2, BLK, 128), lambda i: (0, 0, i, 0))
    d_spec = pl.BlockSpec((2, BLK, 128), lambda i: (0, i, 0))
    y_spec = pl.BlockSpec((BLK, 128), lambda i: (i, 0))
    st_spec = pl.BlockSpec((8, 128), lambda i: (0, 0))
    return pl.pallas_call(
        _l2fin_body,
        grid=(GRID,),
        in_specs=[a_spec, d_spec, y_spec],
        out_specs=[y_spec, st_spec],
        out_shape=[jax.ShapeDtypeStruct((N, 128), _f32),
                   jax.ShapeDtypeStruct((8, 128), _f32)],
    )


def _pool_body(y_ref, s_ref, st_ref, p_ref, b_ref, o_ref, acc):
    i = pl.program_id(0)
    st = st_ref[...]
    m = st[0:1] / float(N)
    var = st[1:2] / float(N) - m * m
    scale = p_ref[0:1] * lax.rsqrt(var + 1e-5)
    h = (y_ref[...] - m) * scale + p_ref[1:2] + s_ref[...] + p_ref[2:3]
    lanes = lax.broadcasted_iota(jnp.int32, (BLK, 128), 1)
    oh = (b_ref[...] == lanes).astype(_f32)
    cat = jnp.concatenate([h, jnp.ones((BLK, 128), _f32)], axis=1)
    contrib = lax.dot_general(oh, cat, (((0,), (0,)), ((), ())),
                              precision=lax.Precision.HIGHEST,
                              preferred_element_type=_f32)

    @pl.when(i == 0)
    def _():
        acc[...] = contrib

    @pl.when(i > 0)
    def _():
        acc[...] = acc[...] + contrib

    @pl.when(i == GRID - 1)
    def _():
        a = acc[...]
        cnt = jnp.maximum(a[:NG, 128:129], 1.0)
        pooled = a[:NG, :128] / cnt
        mx = jnp.max(pooled, axis=1, keepdims=True)
        z = pooled - mx
        lse = jnp.log(jnp.sum(jnp.exp(z), axis=1, keepdims=True))
        o_ref[...] = z - lse


def _pool():
    y_spec = pl.BlockSpec((BLK, 128), lambda i: (i, 0))
    st_spec = pl.BlockSpec((8, 128), lambda i: (0, 0))
    o_spec = pl.BlockSpec((NG, 128), lambda i: (0, 0))
    return pl.pallas_call(
        _pool_body,
        grid=(GRID,),
        in_specs=[y_spec, y_spec, st_spec, st_spec, y_spec],
        out_specs=o_spec,
        out_shape=jax.ShapeDtypeStruct((NG, 128), _f32),
        scratch_shapes=[pltpu.VMEM((128, 256), _f32)],
    )


def _params(g, be, bp):
    return jnp.concatenate(
        [g[None], be[None], bp[None], jnp.zeros((5, g.shape[0]), _f32)], axis=0)


def kernel(x, W_l0, b_l0, W_r0, W_l1, b_l1, W_r1, W_l2, b_l2, W_r2,
           Wp0, bp0, Wp2, bp2, g0, be0, g1, be1, g2, be2, edge_index, batch):
    del b_l0, b_l1, b_l2
    src = edge_index[0]
    dst = edge_index[1]
    ep = NTILES * EDGES_PER_TILE
    src_p = jnp.concatenate(
        [src, jnp.zeros((ep - E,), jnp.int32)]).reshape(NTILES, NBATCH, K)
    dst_p = jnp.concatenate(
        [dst, jnp.full((ep - E,), DUMP_ROW, jnp.int32)]).reshape(
            NTILES, NBATCH, K)

    ones_rows = jnp.ones((N, 128), _f32)
    a0 = _make_segsum(3)(x[:, :128], x[:, 128:], ones_rows, src_p, dst_p)
    d_parts = a0[2]
    y0, s0, st0 = _sage_a(2, 256, 512, True)(a0, d_parts, x, W_l0, W_r0, Wp0)
    h0 = _bn_b(512)(y0, s0, st0, _params(g0, be0, bp0))

    a1 = _make_segsum(4)(h0[:, :128], h0[:, 128:256], h0[:, 256:384],
                         h0[:, 384:], src_p, dst_p)
    z1 = jnp.zeros((512,), _f32)
    y1, st1 = _sage_a(4, 512, 512, False)(a1, d_parts, h0, W_l1, W_r1)
    h1 = _bn_b(512)(y1, h0, st1, _params(g1, be1, z1))

    yl2, yr2, s2 = _l2pre()(h1, W_l2, W_r2, Wp2)
    a2 = _make_segsum(1)(yl2, src_p, dst_p)
    y2, st2 = _l2fin()(a2, d_parts, yr2)
    bb = jnp.broadcast_to(batch[:, None], (N, 128))
    return _pool()(y2, s2, st2, _params(g2, be2, bp2), bb)


# speedup vs baseline: 1.9614x; 1.0726x over previous
import functools

import jax
import jax.numpy as jnp
from jax import lax
from jax.experimental import pallas as pl
from jax.experimental.pallas import tpu as pltpu
from jax.experimental.pallas import tpu_sc as plsc

N = 10000
E = 160000
NG = 64
NTILES = 32
EDGES_PER_TILE = 5120
NBATCH = 40
K = 128
NROWS = 10240
ROWS_PER_SUB = NROWS // 16
DUMP_ROW = N
BLK = 1000
GRID = N // BLK

_f32 = jnp.float32


def _dotT(a, w):
    return lax.dot_general(a, w, (((1,), (1,)), ((), ())),
                           precision=lax.Precision.HIGHEST,
                           preferred_element_type=_f32)


def _make_segsum(nch):
    C = 128
    mesh = plsc.VectorSubcoreMesh(core_axis_name="c", subcore_axis_name="s")
    scratch = [
        pltpu.VMEM((NBATCH, K), jnp.int32),
        pltpu.VMEM((NBATCH, K), jnp.int32),
        pltpu.VMEM((K, C), _f32),
        pltpu.VMEM((K, C), _f32),
        pltpu.VMEM((16, C), _f32),
        pltpu.VMEM_SHARED((NROWS, C), _f32),
        pltpu.SemaphoreType.DMA,
        pltpu.SemaphoreType.DMA,
    ]

    def body(*refs):
        xs = refs[:nch]
        (src_hbm, dst_hbm, out_hbm,
         src_v, dst_v, rows0, rows1, zbuf, acc, sem0, sem1) = refs[nch:]
        cid = lax.axis_index("c")
        sid = lax.axis_index("s")
        wid = cid * 16 + sid
        pltpu.sync_copy(src_hbm.at[wid], src_v)
        pltpu.sync_copy(dst_hbm.at[wid], dst_v)
        for j in range(16):
            for k2 in range(C // 16):
                zbuf[j, pl.ds(k2 * 16, 16)] = jnp.zeros((16,), _f32)
        base = sid * ROWS_PER_SUB
        for c in range(nch):
            def zrow(r, carry):
                pltpu.sync_copy(zbuf, acc.at[pl.ds(base + r * 16, 16)])
                return carry
            lax.fori_loop(0, ROWS_PER_SUB // 16, zrow, 0)
            plsc.subcore_barrier()

            pltpu.async_copy(xs[c].at[src_v.at[0]], rows0, sem0)

            def step2(j, carry):
                b0 = j * 2
                pltpu.make_async_copy(xs[c].at[src_v.at[b0]],
                                      rows0, sem0).wait()
                pltpu.async_copy(xs[c].at[src_v.at[b0 + 1]], rows1, sem1)
                pltpu.sync_copy(rows0, acc.at[dst_v.at[b0]], add=True)
                pltpu.make_async_copy(xs[c].at[src_v.at[b0 + 1]],
                                      rows1, sem1).wait()

                @pl.when(b0 + 2 < NBATCH)
                def _():
                    pltpu.async_copy(xs[c].at[src_v.at[b0 + 2]], rows0, sem0)

                pltpu.sync_copy(rows1, acc.at[dst_v.at[b0 + 1]], add=True)
                return carry
            lax.fori_loop(0, NBATCH // 2, step2, 0)
            plsc.subcore_barrier()
            pltpu.sync_copy(acc.at[pl.ds(base, ROWS_PER_SUB)],
                            out_hbm.at[c, cid, pl.ds(base, ROWS_PER_SUB)])
            plsc.subcore_barrier()

    return pl.kernel(
        body,
        out_type=jax.ShapeDtypeStruct((nch, 2, NROWS, 128), _f32),
        mesh=mesh,
        scratch_types=scratch,
    )


def _stats_update(st_ref, y, i):
    srow = jnp.sum(y, axis=0, keepdims=True)
    sq = jnp.sum(y * y, axis=0, keepdims=True)
    upd = jnp.concatenate(
        [srow, sq, jnp.zeros((6, y.shape[1]), _f32)], axis=0)

    @pl.when(i == 0)
    def _():
        st_ref[...] = upd

    @pl.when(i > 0)
    def _():
        st_ref[...] = st_ref[...] + upd


def _dinv_col(d_ref):
    dcol = (d_ref[0] + d_ref[1])[:, :1]
    return 1.0 / jnp.maximum(dcol, 1.0)


def _sage_a_body(nch, with_skip, a_ref, d_ref, x_ref, wl_ref, wr_ref, *rest):
    if with_skip:
        wp_ref, y_ref, s_ref, st_ref = rest
    else:
        y_ref, st_ref = rest
    i = pl.program_id(0)
    parts = [a_ref[c, 0] + a_ref[c, 1] for c in range(nch)]
    p = parts[0] if nch == 1 else jnp.concatenate(parts, axis=-1)
    mean = p * _dinv_col(d_ref)
    xb = x_ref[...]
    y = _dotT(mean, wl_ref[...]) + _dotT(xb, wr_ref[...])
    y_ref[...] = y
    if with_skip:
        s_ref[...] = _dotT(xb, wp_ref[...])
    _stats_update(st_ref, y, i)


def _sage_a(nch, Din, Dout, with_skip):
    a_spec = pl.BlockSpec((nch, 2, BLK, 128), lambda i: (0, 0, i, 0))
    d_spec = pl.BlockSpec((2, BLK, 128), lambda i: (0, i, 0))
    x_spec = pl.BlockSpec((BLK, Din), lambda i: (i, 0))
    w_spec = pl.BlockSpec((Dout, Din), lambda i: (0, 0))
    y_spec = pl.BlockSpec((BLK, Dout), lambda i: (i, 0))
    st_spec = pl.BlockSpec((8, Dout), lambda i: (0, 0))
    in_specs = [a_spec, d_spec, x_spec, w_spec, w_spec]
    out_shapes = [jax.ShapeDtypeStruct((N, Dout), _f32),
                  jax.ShapeDtypeStruct((8, Dout), _f32)]
    out_specs = [y_spec, st_spec]
    if with_skip:
        in_specs.append(w_spec)
        out_shapes.insert(1, jax.ShapeDtypeStruct((N, Dout), _f32))
        out_specs.insert(1, y_spec)
    return pl.pallas_call(
        functools.partial(_sage_a_body, nch, with_skip),
        grid=(GRID,),
        in_specs=in_specs,
        out_specs=out_specs,
        out_shape=out_shapes,
    )


def _bn_b_body(relu, y_ref, s_ref, st_ref, p_ref, o_ref):
    st = st_ref[...]
    m = st[0:1] / float(N)
    var = st[1:2] / float(N) - m * m
    scale = p_ref[0:1] * lax.rsqrt(var + 1e-5)
    h = (y_ref[...] - m) * scale + p_ref[1:2] + s_ref[...] + p_ref[2:3]
    o_ref[...] = jnp.maximum(h, 0.0) if relu else h


def _bn_b(Dout, relu=True):
    y_spec = pl.BlockSpec((BLK, Dout), lambda i: (i, 0))
    st_spec = pl.BlockSpec((8, Dout), lambda i: (0, 0))
    return pl.pallas_call(
        functools.partial(_bn_b_body, relu),
        grid=(GRID,),
        in_specs=[y_spec, y_spec, st_spec, st_spec],
        out_specs=y_spec,
        out_shape=jax.ShapeDtypeStruct((N, Dout), _f32),
    )


def _l2pre_body(h_ref, wl_ref, wr_ref, wp_ref, yl_ref, yr_ref, s_ref):
    hb = h_ref[...]
    yl_ref[...] = _dotT(hb, wl_ref[...])
    yr_ref[...] = _dotT(hb, wr_ref[...])
    s_ref[...] = _dotT(hb, wp_ref[...])


def _l2pre():
    h_spec = pl.BlockSpec((BLK, 512), lambda i: (i, 0))
    w_spec = pl.BlockSpec((128, 512), lambda i: (0, 0))
    o_spec = pl.BlockSpec((BLK, 128), lambda i: (i, 0))
    return pl.pallas_call(
        _l2pre_body,
        grid=(GRID,),
        in_specs=[h_spec, w_spec, w_spec, w_spec],
        out_specs=[o_spec, o_spec, o_spec],
        out_shape=[jax.ShapeDtypeStruct((N, 128), _f32)] * 3,
    )


def _l2fin_body(a_ref, d_ref, yr_ref, y_ref, st_ref):
    i = pl.program_id(0)
    y = (a_ref[0, 0] + a_ref[0, 1]) * _dinv_col(d_ref) + yr_ref[...]
    y_ref[...] = y
    _stats_update(st_ref, y, i)


def _l2fin():
    a_spec = pl.BlockSpec((1, 2, BLK, 128), lambda i: (0, 0, i, 0))
    d_spec = pl.BlockSpec((2, BLK, 128), lambda i: (0, i, 0))
    y_spec = pl.BlockSpec((BLK, 128), lambda i: (i, 0))
    st_spec = pl.BlockSpec((8, 128), lambda i: (0, 0))
    return pl.pallas_call(
        _l2fin_body,
        grid=(GRID,),
        in_specs=[a_spec, d_spec, y_spec],
        out_specs=[y_spec, st_spec],
        out_shape=[jax.ShapeDtypeStruct((N, 128), _f32),
                   jax.ShapeDtypeStruct((8, 128), _f32)],
    )


def _pool_body(y_ref, s_ref, st_ref, p_ref, b_ref, o_ref, acc):
    i = pl.program_id(0)
    st = st_ref[...]
    m = st[0:1] / float(N)
    var = st[1:2] / float(N) - m * m
    scale = p_ref[0:1] * lax.rsqrt(var + 1e-5)
    h = (y_ref[...] - m) * scale + p_ref[1:2] + s_ref[...] + p_ref[2:3]
    lanes = lax.broadcasted_iota(jnp.int32, (BLK, 128), 1)
    oh = (b_ref[...] == lanes).astype(_f32)
    cat = jnp.concatenate([h, jnp.ones((BLK, 128), _f32)], axis=1)
    contrib = lax.dot_general(oh, cat, (((0,), (0,)), ((), ())),
                              precision=lax.Precision.HIGHEST,
                              preferred_element_type=_f32)

    @pl.when(i == 0)
    def _():
        acc[...] = contrib

    @pl.when(i > 0)
    def _():
        acc[...] = acc[...] + contrib

    @pl.when(i == GRID - 1)
    def _():
        a = acc[...]
        cnt = jnp.maximum(a[:NG, 128:129], 1.0)
        pooled = a[:NG, :128] / cnt
        mx = jnp.max(pooled, axis=1, keepdims=True)
        z = pooled - mx
        lse = jnp.log(jnp.sum(jnp.exp(z), axis=1, keepdims=True))
        o_ref[...] = z - lse


def _pool():
    y_spec = pl.BlockSpec((BLK, 128), lambda i: (i, 0))
    st_spec = pl.BlockSpec((8, 128), lambda i: (0, 0))
    o_spec = pl.BlockSpec((NG, 128), lambda i: (0, 0))
    return pl.pallas_call(
        _pool_body,
        grid=(GRID,),
        in_specs=[y_spec, y_spec, st_spec, st_spec, y_spec],
        out_specs=o_spec,
        out_shape=jax.ShapeDtypeStruct((NG, 128), _f32),
        scratch_shapes=[pltpu.VMEM((128, 256), _f32)],
    )


def _params(g, be, bp):
    return jnp.concatenate(
        [g[None], be[None], bp[None], jnp.zeros((5, g.shape[0]), _f32)], axis=0)


def kernel(x, W_l0, b_l0, W_r0, W_l1, b_l1, W_r1, W_l2, b_l2, W_r2,
           Wp0, bp0, Wp2, bp2, g0, be0, g1, be1, g2, be2, edge_index, batch):
    del b_l0, b_l1, b_l2
    src = edge_index[0]
    dst = edge_index[1]
    ep = NTILES * EDGES_PER_TILE
    src_p = jnp.concatenate(
        [src, jnp.zeros((ep - E,), jnp.int32)]).reshape(NTILES, NBATCH, K)
    pad_dst = DUMP_ROW + (jnp.arange(ep - E, dtype=jnp.int32) % (NROWS - N))
    dst_p = jnp.concatenate([dst, pad_dst]).reshape(NTILES, NBATCH, K)

    ones_rows = jnp.ones((N, 128), _f32)
    a0 = _make_segsum(3)(x[:, :128], x[:, 128:], ones_rows, src_p, dst_p)
    d_parts = a0[2]
    y0, s0, st0 = _sage_a(2, 256, 512, True)(a0, d_parts, x, W_l0, W_r0, Wp0)
    h0 = _bn_b(512)(y0, s0, st0, _params(g0, be0, bp0))

    a1 = _make_segsum(4)(h0[:, :128], h0[:, 128:256], h0[:, 256:384],
                         h0[:, 384:], src_p, dst_p)
    z1 = jnp.zeros((512,), _f32)
    y1, st1 = _sage_a(4, 512, 512, False)(a1, d_parts, h0, W_l1, W_r1)
    h1 = _bn_b(512)(y1, h0, st1, _params(g1, be1, z1))

    yl2, yr2, s2 = _l2pre()(h1, W_l2, W_r2, Wp2)
    a2 = _make_segsum(1)(yl2, src_p, dst_p)
    y2, st2 = _l2fin()(a2, d_parts, yr2)
    bb = jnp.broadcast_to(batch[:, None], (N, 128))
    return _pool()(y2, s2, st2, _params(g2, be2, bp2), bb)


# speedup vs baseline: 2.1512x; 1.0968x over previous
import functools

import jax
import jax.numpy as jnp
from jax import lax
from jax.experimental import pallas as pl
from jax.experimental.pallas import tpu as pltpu
from jax.experimental.pallas import tpu_sc as plsc

N = 10000
E = 160000
NG = 64
NTILES = 32
EDGES_PER_TILE = 5120
NBATCH = 40
K = 128
NROWS = 10240
ROWS_PER_SUB = NROWS // 16
DUMP_ROW = N
BLK = 1000
GRID = N // BLK

_f32 = jnp.float32


def _dotT(a, w):
    return lax.dot_general(a, w, (((1,), (1,)), ((), ())),
                           precision=lax.Precision.HIGHEST,
                           preferred_element_type=_f32)


def _make_segsum(nch, const_last=False):
    C = 128
    mesh = plsc.VectorSubcoreMesh(core_axis_name="c", subcore_axis_name="s")
    scratch = [
        pltpu.VMEM((NBATCH, K), jnp.int32),
        pltpu.VMEM((NBATCH, K), jnp.int32),
        pltpu.VMEM((K, C), _f32),
        pltpu.VMEM((K, C), _f32),
        pltpu.VMEM((16, C), _f32),
        pltpu.VMEM_SHARED((NROWS, C), _f32),
        pltpu.SemaphoreType.DMA,
        pltpu.SemaphoreType.DMA,
    ]

    def body(*refs):
        xs = refs[:nch]
        (src_hbm, dst_hbm, out_hbm,
         src_v, dst_v, rows0, rows1, zbuf, acc, sem0, sem1) = refs[nch:]
        cid = lax.axis_index("c")
        sid = lax.axis_index("s")
        wid = cid * 16 + sid
        pltpu.sync_copy(src_hbm.at[wid], src_v)
        pltpu.sync_copy(dst_hbm.at[wid], dst_v)
        for j in range(16):
            for k2 in range(C // 16):
                zbuf[j, pl.ds(k2 * 16, 16)] = jnp.zeros((16,), _f32)
        base = sid * ROWS_PER_SUB
        for c in range(nch):
            def zrow(r, carry):
                pltpu.sync_copy(zbuf, acc.at[pl.ds(base + r * 16, 16)])
                return carry
            lax.fori_loop(0, ROWS_PER_SUB // 16, zrow, 0)
            plsc.subcore_barrier()

            if const_last and c == nch - 1:
                pltpu.sync_copy(xs[c].at[pl.ds(0, K)], rows0)

                def stepc(b, carry):
                    pltpu.sync_copy(rows0, acc.at[dst_v.at[b]], add=True)
                    return carry
                lax.fori_loop(0, NBATCH, stepc, 0)
                plsc.subcore_barrier()
                pltpu.sync_copy(acc.at[pl.ds(base, ROWS_PER_SUB)],
                                out_hbm.at[c, cid, pl.ds(base, ROWS_PER_SUB)])
                plsc.subcore_barrier()
                continue

            pltpu.async_copy(xs[c].at[src_v.at[0]], rows0, sem0)

            def step2(j, carry):
                b0 = j * 2
                pltpu.make_async_copy(xs[c].at[src_v.at[b0]],
                                      rows0, sem0).wait()
                pltpu.async_copy(xs[c].at[src_v.at[b0 + 1]], rows1, sem1)
                pltpu.sync_copy(rows0, acc.at[dst_v.at[b0]], add=True)
                pltpu.make_async_copy(xs[c].at[src_v.at[b0 + 1]],
                                      rows1, sem1).wait()

                @pl.when(b0 + 2 < NBATCH)
                def _():
                    pltpu.async_copy(xs[c].at[src_v.at[b0 + 2]], rows0, sem0)

                pltpu.sync_copy(rows1, acc.at[dst_v.at[b0 + 1]], add=True)
                return carry
            lax.fori_loop(0, NBATCH // 2, step2, 0)
            plsc.subcore_barrier()
            pltpu.sync_copy(acc.at[pl.ds(base, ROWS_PER_SUB)],
                            out_hbm.at[c, cid, pl.ds(base, ROWS_PER_SUB)])
            plsc.subcore_barrier()

    return pl.kernel(
        body,
        out_type=jax.ShapeDtypeStruct((nch, 2, NROWS, 128), _f32),
        mesh=mesh,
        scratch_types=scratch,
    )


def _stats_update(st_ref, y, i):
    srow = jnp.sum(y, axis=0, keepdims=True)
    sq = jnp.sum(y * y, axis=0, keepdims=True)
    upd = jnp.concatenate(
        [srow, sq, jnp.zeros((6, y.shape[1]), _f32)], axis=0)

    @pl.when(i == 0)
    def _():
        st_ref[...] = upd

    @pl.when(i > 0)
    def _():
        st_ref[...] = st_ref[...] + upd


def _dinv_col(d_ref):
    dcol = (d_ref[0] + d_ref[1])[:, :1]
    return 1.0 / jnp.maximum(dcol, 1.0)


def _sage_a_body(nch, with_skip, a_ref, d_ref, x_ref, wl_ref, wr_ref, *rest):
    if with_skip:
        wp_ref, y_ref, s_ref, st_ref = rest
    else:
        y_ref, st_ref = rest
    i = pl.program_id(0)
    parts = [a_ref[c, 0] + a_ref[c, 1] for c in range(nch)]
    p = parts[0] if nch == 1 else jnp.concatenate(parts, axis=-1)
    mean = p * _dinv_col(d_ref)
    xb = x_ref[...]
    y = _dotT(mean, wl_ref[...]) + _dotT(xb, wr_ref[...])
    y_ref[...] = y
    if with_skip:
        s_ref[...] = _dotT(xb, wp_ref[...])
    _stats_update(st_ref, y, i)


def _sage_a(nch, Din, Dout, with_skip):
    a_spec = pl.BlockSpec((nch, 2, BLK, 128), lambda i: (0, 0, i, 0))
    d_spec = pl.BlockSpec((2, BLK, 128), lambda i: (0, i, 0))
    x_spec = pl.BlockSpec((BLK, Din), lambda i: (i, 0))
    w_spec = pl.BlockSpec((Dout, Din), lambda i: (0, 0))
    y_spec = pl.BlockSpec((BLK, Dout), lambda i: (i, 0))
    st_spec = pl.BlockSpec((8, Dout), lambda i: (0, 0))
    in_specs = [a_spec, d_spec, x_spec, w_spec, w_spec]
    out_shapes = [jax.ShapeDtypeStruct((N, Dout), _f32),
                  jax.ShapeDtypeStruct((8, Dout), _f32)]
    out_specs = [y_spec, st_spec]
    if with_skip:
        in_specs.append(w_spec)
        out_shapes.insert(1, jax.ShapeDtypeStruct((N, Dout), _f32))
        out_specs.insert(1, y_spec)
    return pl.pallas_call(
        functools.partial(_sage_a_body, nch, with_skip),
        grid=(GRID,),
        in_specs=in_specs,
        out_specs=out_specs,
        out_shape=out_shapes,
    )


def _bn_b_body(relu, y_ref, s_ref, st_ref, p_ref, o_ref):
    st = st_ref[...]
    m = st[0:1] / float(N)
    var = st[1:2] / float(N) - m * m
    scale = p_ref[0:1] * lax.rsqrt(var + 1e-5)
    h = (y_ref[...] - m) * scale + p_ref[1:2] + s_ref[...] + p_ref[2:3]
    o_ref[...] = jnp.maximum(h, 0.0) if relu else h


def _bn_b(Dout, relu=True):
    y_spec = pl.BlockSpec((BLK, Dout), lambda i: (i, 0))
    st_spec = pl.BlockSpec((8, Dout), lambda i: (0, 0))
    return pl.pallas_call(
        functools.partial(_bn_b_body, relu),
        grid=(GRID,),
        in_specs=[y_spec, y_spec, st_spec, st_spec],
        out_specs=y_spec,
        out_shape=jax.ShapeDtypeStruct((N, Dout), _f32),
    )


def _l2pre_body(h_ref, wl_ref, wr_ref, wp_ref, yl_ref, yr_ref, s_ref):
    hb = h_ref[...]
    yl_ref[...] = _dotT(hb, wl_ref[...])
    yr_ref[...] = _dotT(hb, wr_ref[...])
    s_ref[...] = _dotT(hb, wp_ref[...])


def _l2pre():
    h_spec = pl.BlockSpec((BLK, 512), lambda i: (i, 0))
    w_spec = pl.BlockSpec((128, 512), lambda i: (0, 0))
    o_spec = pl.BlockSpec((BLK, 128), lambda i: (i, 0))
    return pl.pallas_call(
        _l2pre_body,
        grid=(GRID,),
        in_specs=[h_spec, w_spec, w_spec, w_spec],
        out_specs=[o_spec, o_spec, o_spec],
        out_shape=[jax.ShapeDtypeStruct((N, 128), _f32)] * 3,
    )


def _l2fin_body(a_ref, d_ref, yr_ref, y_ref, st_ref):
    i = pl.program_id(0)
    y = (a_ref[0, 0] + a_ref[0, 1]) * _dinv_col(d_ref) + yr_ref[...]
    y_ref[...] = y
    _stats_update(st_ref, y, i)


def _l2fin():
    a_spec = pl.BlockSpec((1, 2, BLK, 128), lambda i: (0, 0, i, 0))
    d_spec = pl.BlockSpec((2, BLK, 128), lambda i: (0, i, 0))
    y_spec = pl.BlockSpec((BLK, 128), lambda i: (i, 0))
    st_spec = pl.BlockSpec((8, 128), lambda i: (0, 0))
    return pl.pallas_call(
        _l2fin_body,
        grid=(GRID,),
        in_specs=[a_spec, d_spec, y_spec],
        out_specs=[y_spec, st_spec],
        out_shape=[jax.ShapeDtypeStruct((N, 128), _f32),
                   jax.ShapeDtypeStruct((8, 128), _f32)],
    )


def _pool_body(y_ref, s_ref, st_ref, p_ref, b_ref, o_ref, acc):
    i = pl.program_id(0)
    st = st_ref[...]
    m = st[0:1] / float(N)
    var = st[1:2] / float(N) - m * m
    scale = p_ref[0:1] * lax.rsqrt(var + 1e-5)
    h = (y_ref[...] - m) * scale + p_ref[1:2] + s_ref[...] + p_ref[2:3]
    lanes = lax.broadcasted_iota(jnp.int32, (BLK, 128), 1)
    oh = (b_ref[...] == lanes).astype(_f32)
    cat = jnp.concatenate([h, jnp.ones((BLK, 128), _f32)], axis=1)
    contrib = lax.dot_general(oh, cat, (((0,), (0,)), ((), ())),
                              precision=lax.Precision.HIGHEST,
                              preferred_element_type=_f32)

    @pl.when(i == 0)
    def _():
        acc[...] = contrib

    @pl.when(i > 0)
    def _():
        acc[...] = acc[...] + contrib

    @pl.when(i == GRID - 1)
    def _():
        a = acc[...]
        cnt = jnp.maximum(a[:NG, 128:129], 1.0)
        pooled = a[:NG, :128] / cnt
        mx = jnp.max(pooled, axis=1, keepdims=True)
        z = pooled - mx
        lse = jnp.log(jnp.sum(jnp.exp(z), axis=1, keepdims=True))
        o_ref[...] = z - lse


def _pool():
    y_spec = pl.BlockSpec((BLK, 128), lambda i: (i, 0))
    st_spec = pl.BlockSpec((8, 128), lambda i: (0, 0))
    o_spec = pl.BlockSpec((NG, 128), lambda i: (0, 0))
    return pl.pallas_call(
        _pool_body,
        grid=(GRID,),
        in_specs=[y_spec, y_spec, st_spec, st_spec, y_spec],
        out_specs=o_spec,
        out_shape=jax.ShapeDtypeStruct((NG, 128), _f32),
        scratch_shapes=[pltpu.VMEM((128, 256), _f32)],
    )


def _params(g, be, bp):
    return jnp.concatenate(
        [g[None], be[None], bp[None], jnp.zeros((5, g.shape[0]), _f32)], axis=0)


def kernel(x, W_l0, b_l0, W_r0, W_l1, b_l1, W_r1, W_l2, b_l2, W_r2,
           Wp0, bp0, Wp2, bp2, g0, be0, g1, be1, g2, be2, edge_index, batch):
    del b_l0, b_l1, b_l2
    src = edge_index[0]
    dst = edge_index[1]
    ep = NTILES * EDGES_PER_TILE
    src_p = jnp.concatenate(
        [src, jnp.zeros((ep - E,), jnp.int32)]).reshape(NTILES, NBATCH, K)
    pad_dst = DUMP_ROW + (jnp.arange(ep - E, dtype=jnp.int32) % (NROWS - N))
    dst_p = jnp.concatenate([dst, pad_dst]).reshape(NTILES, NBATCH, K)

    ones_rows = jnp.ones((K, 128), _f32)
    a0 = _make_segsum(3, const_last=True)(x[:, :128], x[:, 128:], ones_rows,
                                          src_p, dst_p)
    d_parts = a0[2]
    y0, s0, st0 = _sage_a(2, 256, 512, True)(a0, d_parts, x, W_l0, W_r0, Wp0)
    h0 = _bn_b(512)(y0, s0, st0, _params(g0, be0, bp0))

    a1 = _make_segsum(4)(h0[:, :128], h0[:, 128:256], h0[:, 256:384],
                         h0[:, 384:], src_p, dst_p)
    z1 = jnp.zeros((512,), _f32)
    y1, st1 = _sage_a(4, 512, 512, False)(a1, d_parts, h0, W_l1, W_r1)
    h1 = _bn_b(512)(y1, h0, st1, _params(g1, be1, z1))

    yl2, yr2, s2 = _l2pre()(h1, W_l2, W_r2, Wp2)
    a2 = _make_segsum(1)(yl2, src_p, dst_p)
    y2, st2 = _l2fin()(a2, d_parts, yr2)
    bb = jnp.broadcast_to(batch[:, None], (N, 128))
    return _pool()(y2, s2, st2, _params(g2, be2, bp2), bb)


# speedup vs baseline: 2.1681x; 1.0078x over previous
import functools

import jax
import jax.numpy as jnp
from jax import lax
from jax.experimental import pallas as pl
from jax.experimental.pallas import tpu as pltpu
from jax.experimental.pallas import tpu_sc as plsc

N = 10000
E = 160000
NG = 64
NTILES = 32
EDGES_PER_TILE = 5120
NBATCH = 40
K = 128
NROWS = 10240
ROWS_PER_SUB = NROWS // 16
DUMP_ROW = N
BLK = 1000
GRID = N // BLK

_f32 = jnp.float32


def _dotT(a, w):
    return lax.dot_general(a, w, (((1,), (1,)), ((), ())),
                           precision=lax.Precision.HIGHEST,
                           preferred_element_type=_f32)


def _make_segsum(nch, const_last=False):
    C = 128
    mesh = plsc.VectorSubcoreMesh(core_axis_name="c", subcore_axis_name="s")
    scratch = [
        pltpu.VMEM((NBATCH, K), jnp.int32),
        pltpu.VMEM((NBATCH, K), jnp.int32),
        pltpu.VMEM((K, C), _f32),
        pltpu.VMEM((K, C), _f32),
        pltpu.VMEM((16, C), _f32),
        pltpu.VMEM_SHARED((NROWS, C), _f32),
        pltpu.SemaphoreType.DMA,
        pltpu.SemaphoreType.DMA,
    ]

    def body(*refs):
        xs = refs[:nch]
        (src_hbm, dst_hbm, out_hbm,
         src_v, dst_v, rows0, rows1, zbuf, acc, sem0, sem1) = refs[nch:]
        cid = lax.axis_index("c")
        sid = lax.axis_index("s")
        wid = cid * 16 + sid
        pltpu.sync_copy(src_hbm.at[wid], src_v)
        pltpu.sync_copy(dst_hbm.at[wid], dst_v)
        for j in range(16):
            for k2 in range(C // 16):
                zbuf[j, pl.ds(k2 * 16, 16)] = jnp.zeros((16,), _f32)
        base = sid * ROWS_PER_SUB
        for c in range(nch):
            def zrow(r, carry):
                pltpu.sync_copy(zbuf, acc.at[pl.ds(base + r * 16, 16)])
                return carry
            lax.fori_loop(0, ROWS_PER_SUB // 16, zrow, 0)
            plsc.subcore_barrier()

            if const_last and c == nch - 1:
                pltpu.sync_copy(xs[c].at[pl.ds(0, K)], rows0)

                def stepc(b, carry):
                    pltpu.sync_copy(rows0, acc.at[dst_v.at[b]], add=True)
                    return carry
                lax.fori_loop(0, NBATCH, stepc, 0)
                plsc.subcore_barrier()
                pltpu.sync_copy(acc.at[pl.ds(base, ROWS_PER_SUB)],
                                out_hbm.at[c, cid, pl.ds(base, ROWS_PER_SUB)])
                plsc.subcore_barrier()
                continue

            pltpu.async_copy(xs[c].at[src_v.at[0]], rows0, sem0)

            def step2(j, carry):
                b0 = j * 2
                pltpu.make_async_copy(xs[c].at[src_v.at[b0]],
                                      rows0, sem0).wait()
                pltpu.async_copy(xs[c].at[src_v.at[b0 + 1]], rows1, sem1)
                pltpu.sync_copy(rows0, acc.at[dst_v.at[b0]], add=True)
                pltpu.make_async_copy(xs[c].at[src_v.at[b0 + 1]],
                                      rows1, sem1).wait()

                @pl.when(b0 + 2 < NBATCH)
                def _():
                    pltpu.async_copy(xs[c].at[src_v.at[b0 + 2]], rows0, sem0)

                pltpu.sync_copy(rows1, acc.at[dst_v.at[b0 + 1]], add=True)
                return carry
            lax.fori_loop(0, NBATCH // 2, step2, 0)
            plsc.subcore_barrier()
            pltpu.sync_copy(acc.at[pl.ds(base, ROWS_PER_SUB)],
                            out_hbm.at[c, cid, pl.ds(base, ROWS_PER_SUB)])
            plsc.subcore_barrier()

    return pl.kernel(
        body,
        out_type=jax.ShapeDtypeStruct((nch, 2, NROWS, 128), _f32),
        mesh=mesh,
        scratch_types=scratch,
    )


def _stats_update(st_ref, y, i):
    srow = jnp.sum(y, axis=0, keepdims=True)
    sq = jnp.sum(y * y, axis=0, keepdims=True)
    upd = jnp.concatenate(
        [srow, sq, jnp.zeros((6, y.shape[1]), _f32)], axis=0)

    @pl.when(i == 0)
    def _():
        st_ref[...] = upd

    @pl.when(i > 0)
    def _():
        st_ref[...] = st_ref[...] + upd


def _dinv_col(d_ref):
    dcol = (d_ref[0] + d_ref[1])[:, :1]
    return 1.0 / jnp.maximum(dcol, 1.0)


def _sage_a_body(nch, with_skip, a_ref, d_ref, x_ref, wl_ref, wr_ref, *rest):
    if with_skip:
        wp_ref, y_ref, s_ref, st_ref = rest
    else:
        y_ref, st_ref = rest
    i = pl.program_id(0)
    parts = [a_ref[c, 0] + a_ref[c, 1] for c in range(nch)]
    p = parts[0] if nch == 1 else jnp.concatenate(parts, axis=-1)
    mean = p * _dinv_col(d_ref)
    xb = x_ref[...]
    y = _dotT(mean, wl_ref[...]) + _dotT(xb, wr_ref[...])
    y_ref[...] = y
    if with_skip:
        s_ref[...] = _dotT(xb, wp_ref[...])
    _stats_update(st_ref, y, i)


def _sage_a(nch, Din, Dout, with_skip):
    a_spec = pl.BlockSpec((nch, 2, BLK, 128), lambda i: (0, 0, i, 0))
    d_spec = pl.BlockSpec((2, BLK, 128), lambda i: (0, i, 0))
    x_spec = pl.BlockSpec((BLK, Din), lambda i: (i, 0))
    w_spec = pl.BlockSpec((Dout, Din), lambda i: (0, 0))
    y_spec = pl.BlockSpec((BLK, Dout), lambda i: (i, 0))
    st_spec = pl.BlockSpec((8, Dout), lambda i: (0, 0))
    in_specs = [a_spec, d_spec, x_spec, w_spec, w_spec]
    out_shapes = [jax.ShapeDtypeStruct((N, Dout), _f32),
                  jax.ShapeDtypeStruct((8, Dout), _f32)]
    out_specs = [y_spec, st_spec]
    if with_skip:
        in_specs.append(w_spec)
        out_shapes.insert(1, jax.ShapeDtypeStruct((N, Dout), _f32))
        out_specs.insert(1, y_spec)
    return pl.pallas_call(
        functools.partial(_sage_a_body, nch, with_skip),
        grid=(GRID,),
        in_specs=in_specs,
        out_specs=out_specs,
        out_shape=out_shapes,
    )


def _bn_b_body(relu, y_ref, s_ref, st_ref, p_ref, o_ref):
    st = st_ref[...]
    m = st[0:1] / float(N)
    var = st[1:2] / float(N) - m * m
    scale = p_ref[0:1] * lax.rsqrt(var + 1e-5)
    h = (y_ref[...] - m) * scale + p_ref[1:2] + s_ref[...] + p_ref[2:3]
    o_ref[...] = jnp.maximum(h, 0.0) if relu else h


def _bn_b(Dout, relu=True):
    y_spec = pl.BlockSpec((BLK, Dout), lambda i: (i, 0))
    st_spec = pl.BlockSpec((8, Dout), lambda i: (0, 0))
    return pl.pallas_call(
        functools.partial(_bn_b_body, relu),
        grid=(GRID,),
        in_specs=[y_spec, y_spec, st_spec, st_spec],
        out_specs=y_spec,
        out_shape=jax.ShapeDtypeStruct((N, Dout), _f32),
    )


def _bn1_l2pre_body(y_ref, s_ref, st_ref, p_ref, wl_ref, wr_ref, wp_ref,
                    h_ref, yl_ref, yr_ref, s2_ref):
    st = st_ref[...]
    m = st[0:1] / float(N)
    var = st[1:2] / float(N) - m * m
    scale = p_ref[0:1] * lax.rsqrt(var + 1e-5)
    h = (y_ref[...] - m) * scale + p_ref[1:2] + s_ref[...] + p_ref[2:3]
    hb = jnp.maximum(h, 0.0)
    h_ref[...] = hb
    yl_ref[...] = _dotT(hb, wl_ref[...])
    yr_ref[...] = _dotT(hb, wr_ref[...])
    s2_ref[...] = _dotT(hb, wp_ref[...])


def _bn1_l2pre():
    y_spec = pl.BlockSpec((BLK, 512), lambda i: (i, 0))
    st_spec = pl.BlockSpec((8, 512), lambda i: (0, 0))
    w_spec = pl.BlockSpec((128, 512), lambda i: (0, 0))
    o_spec = pl.BlockSpec((BLK, 128), lambda i: (i, 0))
    return pl.pallas_call(
        _bn1_l2pre_body,
        grid=(GRID,),
        in_specs=[y_spec, y_spec, st_spec, st_spec, w_spec, w_spec, w_spec],
        out_specs=[y_spec, o_spec, o_spec, o_spec],
        out_shape=[jax.ShapeDtypeStruct((N, 512), _f32)] +
                  [jax.ShapeDtypeStruct((N, 128), _f32)] * 3,
    )


def _l2fin_body(a_ref, d_ref, yr_ref, y_ref, st_ref):
    i = pl.program_id(0)
    y = (a_ref[0, 0] + a_ref[0, 1]) * _dinv_col(d_ref) + yr_ref[...]
    y_ref[...] = y
    _stats_update(st_ref, y, i)


def _l2fin():
    a_spec = pl.BlockSpec((1, 2, BLK, 128), lambda i: (0, 0, i, 0))
    d_spec = pl.BlockSpec((2, BLK, 128), lambda i: (0, i, 0))
    y_spec = pl.BlockSpec((BLK, 128), lambda i: (i, 0))
    st_spec = pl.BlockSpec((8, 128), lambda i: (0, 0))
    return pl.pallas_call(
        _l2fin_body,
        grid=(GRID,),
        in_specs=[a_spec, d_spec, y_spec],
        out_specs=[y_spec, st_spec],
        out_shape=[jax.ShapeDtypeStruct((N, 128), _f32),
                   jax.ShapeDtypeStruct((8, 128), _f32)],
    )


def _pool_body(y_ref, s_ref, st_ref, p_ref, b_ref, o_ref, acc):
    i = pl.program_id(0)
    st = st_ref[...]
    m = st[0:1] / float(N)
    var = st[1:2] / float(N) - m * m
    scale = p_ref[0:1] * lax.rsqrt(var + 1e-5)
    h = (y_ref[...] - m) * scale + p_ref[1:2] + s_ref[...] + p_ref[2:3]
    lanes = lax.broadcasted_iota(jnp.int32, (BLK, 128), 1)
    oh = (b_ref[...] == lanes).astype(_f32)
    cat = jnp.concatenate([h, jnp.ones((BLK, 128), _f32)], axis=1)
    contrib = lax.dot_general(oh, cat, (((0,), (0,)), ((), ())),
                              precision=lax.Precision.HIGHEST,
                              preferred_element_type=_f32)

    @pl.when(i == 0)
    def _():
        acc[...] = contrib

    @pl.when(i > 0)
    def _():
        acc[...] = acc[...] + contrib

    @pl.when(i == GRID - 1)
    def _():
        a = acc[...]
        cnt = jnp.maximum(a[:NG, 128:129], 1.0)
        pooled = a[:NG, :128] / cnt
        mx = jnp.max(pooled, axis=1, keepdims=True)
        z = pooled - mx
        lse = jnp.log(jnp.sum(jnp.exp(z), axis=1, keepdims=True))
        o_ref[...] = z - lse


def _pool():
    y_spec = pl.BlockSpec((BLK, 128), lambda i: (i, 0))
    st_spec = pl.BlockSpec((8, 128), lambda i: (0, 0))
    o_spec = pl.BlockSpec((NG, 128), lambda i: (0, 0))
    return pl.pallas_call(
        _pool_body,
        grid=(GRID,),
        in_specs=[y_spec, y_spec, st_spec, st_spec, y_spec],
        out_specs=o_spec,
        out_shape=jax.ShapeDtypeStruct((NG, 128), _f32),
        scratch_shapes=[pltpu.VMEM((128, 256), _f32)],
    )


def _params(g, be, bp):
    return jnp.concatenate(
        [g[None], be[None], bp[None], jnp.zeros((5, g.shape[0]), _f32)], axis=0)


def kernel(x, W_l0, b_l0, W_r0, W_l1, b_l1, W_r1, W_l2, b_l2, W_r2,
           Wp0, bp0, Wp2, bp2, g0, be0, g1, be1, g2, be2, edge_index, batch):
    del b_l0, b_l1, b_l2
    src = edge_index[0]
    dst = edge_index[1]
    ep = NTILES * EDGES_PER_TILE
    src_p = jnp.concatenate(
        [src, jnp.zeros((ep - E,), jnp.int32)]).reshape(NTILES, NBATCH, K)
    pad_dst = DUMP_ROW + (jnp.arange(ep - E, dtype=jnp.int32) % (NROWS - N))
    dst_p = jnp.concatenate([dst, pad_dst]).reshape(NTILES, NBATCH, K)

    ones_rows = jnp.ones((K, 128), _f32)
    a0 = _make_segsum(3, const_last=True)(x[:, :128], x[:, 128:], ones_rows,
                                          src_p, dst_p)
    d_parts = a0[2]
    y0, s0, st0 = _sage_a(2, 256, 512, True)(a0, d_parts, x, W_l0, W_r0, Wp0)
    h0 = _bn_b(512)(y0, s0, st0, _params(g0, be0, bp0))

    a1 = _make_segsum(4)(h0[:, :128], h0[:, 128:256], h0[:, 256:384],
                         h0[:, 384:], src_p, dst_p)
    z1 = jnp.zeros((512,), _f32)
    y1, st1 = _sage_a(4, 512, 512, False)(a1, d_parts, h0, W_l1, W_r1)
    h1, yl2, yr2, s2 = _bn1_l2pre()(y1, h0, st1, _params(g1, be1, z1),
                                    W_l2, W_r2, Wp2)
    del h1
    a2 = _make_segsum(1)(yl2, src_p, dst_p)
    y2, st2 = _l2fin()(a2, d_parts, yr2)
    bb = jnp.broadcast_to(batch[:, None], (N, 128))
    return _pool()(y2, s2, st2, _params(g2, be2, bp2), bb)
